# baseline (speedup 1.0000x reference)
import functools

import jax
import jax.numpy as jnp
from jax.experimental import pallas as pl

_K = 1024
_D = 256
_BN = 256


def _vq_body(z_ref, e_ref, idx_ref, oh_ref, zq_ref):
    z = z_ref[...]
    e = e_ref[...]
    mm = jax.lax.dot_general(
        z, e, (((1,), (1,)), ((), ())),
        preferred_element_type=jnp.float32,
    )
    zsq = jnp.sum(jnp.square(z), axis=1, keepdims=True)
    esq = jnp.sum(jnp.square(e), axis=1)
    d = zsq - 2.0 * mm + esq[None, :]
    m = jnp.min(d, axis=1, keepdims=True)
    iota = jax.lax.broadcasted_iota(jnp.int32, d.shape, 1)
    idx = jnp.min(jnp.where(d == m, iota, _K), axis=1, keepdims=True)
    idx_ref[...] = idx
    oh = jnp.where(iota == idx, jnp.float32(1.0), jnp.float32(0.0))
    oh_ref[...] = oh
    zq_ref[...] = jax.lax.dot_general(
        oh, e, (((1,), (0,)), ((), ())),
        preferred_element_type=jnp.float32,
    )


@jax.jit
def kernel(z_e, embed):
    n, d_ = z_e.shape
    k = embed.shape[0]
    grid = (n // _BN,)
    idx2d, one_hot, z_q = pl.pallas_call(
        _vq_body,
        grid=grid,
        in_specs=[
            pl.BlockSpec((_BN, d_), lambda i: (i, 0)),
            pl.BlockSpec((k, d_), lambda i: (0, 0)),
        ],
        out_specs=[
            pl.BlockSpec((_BN, 1), lambda i: (i, 0)),
            pl.BlockSpec((_BN, k), lambda i: (i, 0)),
            pl.BlockSpec((_BN, d_), lambda i: (i, 0)),
        ],
        out_shape=[
            jax.ShapeDtypeStruct((n, 1), jnp.int32),
            jax.ShapeDtypeStruct((n, k), jnp.float32),
            jax.ShapeDtypeStruct((n, d_), jnp.float32),
        ],
    )(z_e, embed)
    return z_q, idx2d.reshape(n), one_hot


# speedup vs baseline: 1.0080x; 1.0080x over previous
import functools

import jax
import jax.numpy as jnp
from jax.experimental import pallas as pl

_K = 1024
_D = 256
_BN = 256


def _vq_body(z_ref, e_ref, esq_ref, iota_ref, idx_ref, oh_ref, zq_ref):
    z = z_ref[...]
    e = e_ref[...]
    mm = jax.lax.dot_general(
        z, e, (((1,), (1,)), ((), ())),
        preferred_element_type=jnp.float32,
    )
    zsq = jnp.sum(jnp.square(z), axis=1, keepdims=True)
    d = zsq - 2.0 * mm + esq_ref[...]
    m = jnp.min(d, axis=1, keepdims=True)
    iota = iota_ref[...]
    idxf = jnp.min(
        jnp.where(d == m, iota, jnp.float32(_K)), axis=1, keepdims=True
    )
    idx_ref[...] = idxf.astype(jnp.int32)
    oh = jnp.where(iota == idxf, jnp.float32(1.0), jnp.float32(0.0))
    oh_ref[...] = oh
    zq_ref[...] = jax.lax.dot_general(
        oh.astype(jnp.bfloat16), e.astype(jnp.bfloat16),
        (((1,), (0,)), ((), ())),
        preferred_element_type=jnp.float32,
    )


@jax.jit
def kernel(z_e, embed):
    n, d_ = z_e.shape
    k = embed.shape[0]
    esq = jnp.sum(jnp.square(embed), axis=1)[None, :]
    iota_f = jnp.arange(k, dtype=jnp.float32)[None, :]
    grid = (n // _BN,)
    idx2d, one_hot, z_q = pl.pallas_call(
        _vq_body,
        grid=grid,
        in_specs=[
            pl.BlockSpec((_BN, d_), lambda i: (i, 0)),
            pl.BlockSpec((k, d_), lambda i: (0, 0)),
            pl.BlockSpec((1, k), lambda i: (0, 0)),
            pl.BlockSpec((1, k), lambda i: (0, 0)),
        ],
        out_specs=[
            pl.BlockSpec((_BN, 1), lambda i: (i, 0)),
            pl.BlockSpec((_BN, k), lambda i: (i, 0)),
            pl.BlockSpec((_BN, d_), lambda i: (i, 0)),
        ],
        out_shape=[
            jax.ShapeDtypeStruct((n, 1), jnp.int32),
            jax.ShapeDtypeStruct((n, k), jnp.float32),
            jax.ShapeDtypeStruct((n, d_), jnp.float32),
        ],
    )(z_e, embed, esq, iota_f)
    return z_q, idx2d.reshape(n), one_hot
